# baseline (speedup 1.0000x reference)
import math

import jax
import jax.numpy as jnp
from jax.experimental import pallas as pl
from jax.experimental.pallas import tpu as pltpu

_THR = math.log(0.5)
_TCHUNK = 512


def _body(alpha_ref, blank_ref, mask_ref, out_ref, num_ref, den_ref,
          cnt_ref):
    i = pl.program_id(0)
    j = pl.program_id(1)
    nj = pl.num_programs(1)

    @pl.when(jnp.logical_and(i == 0, j == 0))
    def _init():
        num_ref[0] = 0.0
        den_ref[0] = 0.0

    @pl.when(j == 0)
    def _row_init():
        cnt_ref[0] = 0.0

    x = blank_ref[0]
    tc, l = x.shape
    mvec = mask_ref[i, pl.ds(j * tc, tc)]
    lane = jax.lax.broadcasted_iota(jnp.int32, (tc, l), 1)
    pred = ((1.0 - x) > _THR) & (lane == 0) & (mvec[:, None] != 0.0)
    cnt_ref[0] += jnp.sum(jnp.where(pred, 1.0, 0.0))

    @pl.when(j == nj - 1)
    def _row_fin():
        cnt = cnt_ref[0]
        s = jnp.sum(alpha_ref[i, :])
        num_ref[0] += jnp.abs(s - 1.0) * cnt
        den_ref[0] += jnp.where(cnt > 0.5, 1.0, 0.0)

    @pl.when(jnp.logical_and(i == pl.num_programs(0) - 1, j == nj - 1))
    def _fin():
        n = num_ref[0]
        d = den_ref[0]
        out_ref[:, :] = jnp.where(d > 0.0, n / d, 0.0)[None, None]


def kernel(alpha, ctc_log_probs, mask):
    b, t = alpha.shape
    nj = t // _TCHUNK
    out = pl.pallas_call(
        _body,
        grid=(b, nj),
        in_specs=[
            pl.BlockSpec((b, t), lambda i, j: (0, 0)),
            pl.BlockSpec((1, _TCHUNK, 128), lambda i, j: (i, j, 0)),
            pl.BlockSpec((b, t), lambda i, j: (0, 0)),
        ],
        out_specs=pl.BlockSpec((1, 1), lambda i, j: (0, 0)),
        out_shape=jax.ShapeDtypeStruct((1, 1), jnp.float32),
        scratch_shapes=[
            pltpu.SMEM((1,), jnp.float32),
            pltpu.SMEM((1,), jnp.float32),
            pltpu.SMEM((1,), jnp.float32),
        ],
    )(alpha, ctc_log_probs, mask)
    return out[0, 0]


# speedup vs baseline: 3.3920x; 3.3920x over previous
import math

import jax
import jax.numpy as jnp
from jax.experimental import pallas as pl
from jax.experimental.pallas import tpu as pltpu

_SPIKE_THRESHOLD = math.log(0.5)


def _body(alpha_ref, blank_ref, mask_ref, out_ref, num_ref, den_ref):
    i = pl.program_id(0)

    @pl.when(i == 0)
    def _init():
        num_ref[0] = 0.0
        den_ref[0] = 0.0

    for r in range(2):
        blank = blank_ref[r]
        t, l = blank.shape
        lane = jax.lax.broadcasted_iota(jnp.int32, (t, l), 1)
        trig = ((1.0 - blank) > _SPIKE_THRESHOLD) & (lane == 0)
        spike = trig & (mask_ref[2 * i + r, :][:, None] != 0.0)
        cnt = jnp.sum(spike.astype(jnp.float32))
        s = jnp.sum(alpha_ref[2 * i + r, :])
        num_ref[0] += jnp.abs(s - 1.0) * cnt
        den_ref[0] += jnp.where(cnt > 0.5, 1.0, 0.0)

    @pl.when(i == pl.num_programs(0) - 1)
    def _fin():
        n = num_ref[0]
        d = den_ref[0]
        out_ref[:, :] = jnp.where(d > 0.0, n / d, 0.0)[None, None]


def kernel(alpha, ctc_log_probs, mask):
    b, t = alpha.shape
    out = pl.pallas_call(
        _body,
        grid=(b // 2,),
        in_specs=[
            pl.BlockSpec((b, t), lambda i: (0, 0)),
            pl.BlockSpec((2, t, 128), lambda i: (i, 0, 0)),
            pl.BlockSpec((b, t), lambda i: (0, 0)),
        ],
        out_specs=pl.BlockSpec((1, 1), lambda i: (0, 0)),
        out_shape=jax.ShapeDtypeStruct((1, 1), jnp.float32),
        scratch_shapes=[
            pltpu.SMEM((1,), jnp.float32),
            pltpu.SMEM((1,), jnp.float32),
        ],
    )(alpha, ctc_log_probs, mask)
    return out[0, 0]


# speedup vs baseline: 4.0657x; 1.1986x over previous
import math

import jax
import jax.numpy as jnp
from jax.experimental import pallas as pl
from jax.experimental.pallas import tpu as pltpu

_SPIKE_THRESHOLD = math.log(0.5)


def _body(alpha_ref, blank_ref, mask_ref, out_ref, num_ref, den_ref):
    i = pl.program_id(0)

    @pl.when(i == 0)
    def _init():
        num_ref[0] = 0.0
        den_ref[0] = 0.0

    for r in range(4):
        blank = blank_ref[r]
        t, l = blank.shape
        lane = jax.lax.broadcasted_iota(jnp.int32, (t, l), 1)
        trig = ((1.0 - blank) > _SPIKE_THRESHOLD) & (lane == 0)
        spike = trig & (mask_ref[4 * i + r, :][:, None] != 0.0)
        cnt = jnp.sum(spike.astype(jnp.float32))
        s = jnp.sum(alpha_ref[4 * i + r, :])
        num_ref[0] += jnp.abs(s - 1.0) * cnt
        den_ref[0] += jnp.where(cnt > 0.5, 1.0, 0.0)

    @pl.when(i == pl.num_programs(0) - 1)
    def _fin():
        n = num_ref[0]
        d = den_ref[0]
        out_ref[:, :] = jnp.where(d > 0.0, n / d, 0.0)[None, None]


def kernel(alpha, ctc_log_probs, mask):
    b, t = alpha.shape
    out = pl.pallas_call(
        _body,
        grid=(b // 4,),
        in_specs=[
            pl.BlockSpec((b, t), lambda i: (0, 0)),
            pl.BlockSpec((4, t, 128), lambda i: (i, 0, 0)),
            pl.BlockSpec((b, t), lambda i: (0, 0)),
        ],
        out_specs=pl.BlockSpec((1, 1), lambda i: (0, 0)),
        out_shape=jax.ShapeDtypeStruct((1, 1), jnp.float32),
        scratch_shapes=[
            pltpu.SMEM((1,), jnp.float32),
            pltpu.SMEM((1,), jnp.float32),
        ],
    )(alpha, ctc_log_probs, mask)
    return out[0, 0]
